# Initial kernel scaffold; baseline (speedup 1.0000x reference)
import jax
import jax.numpy as jnp
from jax.experimental import pallas as pl


def kernel(token_ids, weight):
    raise NotImplementedError("write your pallas kernel here")


# speedup vs baseline: 1.5759x; 1.5759x over previous
import functools

import jax
import jax.numpy as jnp
from jax import lax
from jax.experimental import pallas as pl
from jax.experimental.pallas import tpu as pltpu
from jax.experimental.pallas import tpu_sc as plsc

EMBEDDING_DIM = 32

_info = plsc.get_sparse_core_info()
_NC, _NS = _info.num_cores, _info.num_subcores
_NW = _NC * _NS


@functools.lru_cache(maxsize=None)
def _build_gather(total, dim, chunk):
    assert total % _NW == 0
    b_per_w = total // _NW
    assert b_per_w % chunk == 0
    n_chunks = b_per_w // chunk
    mesh = plsc.VectorSubcoreMesh(core_axis_name="c", subcore_axis_name="s")

    @functools.partial(
        pl.kernel,
        mesh=mesh,
        out_type=jax.ShapeDtypeStruct((total, dim), jnp.float32),
        compiler_params=pltpu.CompilerParams(use_tc_tiling_on_sc=False),
        scratch_types=[
            pltpu.VMEM((b_per_w,), jnp.int32),
            pltpu.VMEM((chunk, dim), jnp.float32),
            pltpu.VMEM((chunk, dim), jnp.float32),
            pltpu.SemaphoreType.DMA,
            pltpu.SemaphoreType.DMA,
            pltpu.SemaphoreType.DMA,
            pltpu.SemaphoreType.DMA,
        ],
    )
    def k(table_hbm, idx_hbm, out_hbm, idx_v, buf0, buf1,
          gsem0, gsem1, osem0, osem1):
        wid = lax.axis_index("s") * _NC + lax.axis_index("c")
        base = wid * b_per_w
        pltpu.sync_copy(idx_hbm.at[pl.ds(base, b_per_w)], idx_v)

        bufs = (buf0, buf1)
        gsems = (gsem0, gsem1)
        osems = (osem0, osem1)
        gathers = [None, None]
        out_copies = [None, None]

        gathers[0] = pltpu.async_copy(
            table_hbm.at[idx_v.at[pl.ds(0, chunk)]], buf0, gsem0)
        for c in range(n_chunks):
            slot = c % 2
            nslot = (c + 1) % 2
            if c + 1 < n_chunks:
                if out_copies[nslot] is not None:
                    out_copies[nslot].wait()
                gathers[nslot] = pltpu.async_copy(
                    table_hbm.at[idx_v.at[pl.ds((c + 1) * chunk, chunk)]],
                    bufs[nslot], gsems[nslot])
            gathers[slot].wait()
            out_copies[slot] = pltpu.async_copy(
                bufs[slot], out_hbm.at[pl.ds(base + c * chunk, chunk)],
                osems[slot])
        for oc in out_copies:
            if oc is not None:
                oc.wait()

    return k


def kernel(token_ids, weight):
    batch, fields = token_ids.shape
    total = batch * fields
    flat_idx = token_ids.reshape(total).astype(jnp.int32)
    out = _build_gather(total, EMBEDDING_DIM, 1664)(weight, flat_idx)
    return out.reshape(batch, fields, EMBEDDING_DIM)
